# Initial kernel scaffold; baseline (speedup 1.0000x reference)
import jax
import jax.numpy as jnp
from jax.experimental import pallas as pl


def kernel(x, x1, mask, threshold):
    raise NotImplementedError("write your pallas kernel here")


# speedup vs baseline: 1.0593x; 1.0593x over previous
import jax
import jax.numpy as jnp
from jax.experimental import pallas as pl
from jax.experimental.pallas import tpu as pltpu


def _select_body(t_ref, x_ref, x1_ref, m_ref, o_ref):
    t = t_ref[0]
    o_ref[...] = jnp.where(m_ref[...] >= t, x_ref[...], x1_ref[...])


def kernel(x, x1, mask, threshold):
    B, S, D = x.shape
    x2 = x.reshape(B * S, D)
    x12 = x1.reshape(B * S, D)
    m2 = mask.reshape(B * S, D)
    t = threshold.reshape(1)

    ROWS = 512
    grid = (B * S // ROWS,)
    out = pl.pallas_call(
        _select_body,
        grid=grid,
        in_specs=[
            pl.BlockSpec(memory_space=pltpu.SMEM),
            pl.BlockSpec((ROWS, D), lambda i: (i, 0)),
            pl.BlockSpec((ROWS, D), lambda i: (i, 0)),
            pl.BlockSpec((ROWS, D), lambda i: (i, 0)),
        ],
        out_specs=pl.BlockSpec((ROWS, D), lambda i: (i, 0)),
        out_shape=jax.ShapeDtypeStruct((B * S, D), jnp.float32),
    )(t, x2, x12, m2)
    return out.reshape(B, S, D)
